# baseline (speedup 1.0000x reference)
import functools

import jax
import jax.numpy as jnp
from jax import lax
from jax.experimental import pallas as pl
from jax.experimental.pallas import tpu as pltpu
from jax.experimental.pallas import tpu_sc as plsc

N_FIELDS = 26
FIELD = 50
STRIDE = 51
N_COLS = N_FIELDS * STRIDE
N_BINS_PAD = 1408
BATCH = 1024
N_DENSE = 13
EMB = 32

_NC = 2
_NS = 16
_NW = _NC * _NS
_ROWS_PER_W = BATCH // _NW
_GROUP = 16
_N_GROUPS = _ROWS_PER_W // _GROUP


_HIST_WORDS = _GROUP * N_BINS_PAD
_ZERO_UNROLL = 16


def _zero_hist(hist_v):
    zeros = jnp.zeros((16,), jnp.float32)

    @plsc.parallel_loop(0, N_BINS_PAD // (16 * _ZERO_UNROLL), unroll=2)
    def body(i):
        base = i * (16 * _ZERO_UNROLL)
        for r in range(_GROUP):
            for k in range(_ZERO_UNROLL):
                hist_v[r, pl.ds(base + k * 16, 16)] = zeros


def _scatter_group(block_v, hist_v):
    lanes = lax.iota(jnp.int32, 16)
    row_base = lanes * N_COLS
    ones = jnp.ones((16,), jnp.float32)
    zero16 = jnp.zeros((16,), jnp.int32)

    def field_body(f, _):
        col0 = zero16 + f * STRIDE
        bin0 = zero16 + f * FIELD

        @plsc.parallel_loop(0, FIELD, unroll=10)
        def col_body(j):
            vals = plsc.load_gather(block_v, [lanes, col0 + j])
            plsc.addupdate_scatter(hist_v, [lanes, bin0 + vals], ones)

        return 0
    lax.fori_loop(0, N_FIELDS, field_body, 0)


def _sc_counts_body(sparse_hbm, counts_hbm, block0, block1, hist0, hist1,
                    isem0, isem1, osem0, osem1):
    wid = lax.axis_index("s") * _NC + lax.axis_index("c")
    b0 = wid * _ROWS_PER_W
    b1 = b0 + _GROUP

    in0 = pltpu.async_copy(
        sparse_hbm.at[pl.ds(b0, _GROUP), :], block0, isem0)
    in1 = pltpu.async_copy(
        sparse_hbm.at[pl.ds(b1, _GROUP), :], block1, isem1)
    _zero_hist(hist0)
    _zero_hist(hist1)
    in0.wait()
    _scatter_group(block0, hist0)
    out0 = pltpu.async_copy(
        hist0, counts_hbm.at[pl.ds(b0, _GROUP), :], osem0)
    in1.wait()
    _scatter_group(block1, hist1)
    out1 = pltpu.async_copy(
        hist1, counts_hbm.at[pl.ds(b1, _GROUP), :], osem1)
    out0.wait()
    out1.wait()


def _sc_counts(sparse_2d):
    mesh = plsc.VectorSubcoreMesh(core_axis_name="c", subcore_axis_name="s")
    f = pl.kernel(
        _sc_counts_body,
        mesh=mesh,
        out_type=jax.ShapeDtypeStruct((BATCH, N_BINS_PAD), jnp.float32),
        scratch_types=[
            pltpu.VMEM((_GROUP, N_COLS), jnp.int32),
            pltpu.VMEM((_GROUP, N_COLS), jnp.int32),
            pltpu.VMEM((_GROUP, N_BINS_PAD), jnp.float32),
            pltpu.VMEM((_GROUP, N_BINS_PAD), jnp.float32),
            pltpu.SemaphoreType.DMA,
            pltpu.SemaphoreType.DMA,
            pltpu.SemaphoreType.DMA,
            pltpu.SemaphoreType.DMA,
        ],
        compiler_params=pltpu.CompilerParams(needs_layout_passes=False),
    )
    return f(sparse_2d)


def _tc_body(counts_ref, emb_ref, lin_ref, wf_ref, dense_ref, wd_ref, b_ref,
             out_ref):
    counts = counts_ref[...]
    emb = emb_ref[...]
    sq = jnp.sum(emb * emb, axis=1, keepdims=True)
    wl = lin_ref[...] * wf_ref[...]
    m = jnp.concatenate([emb, sq, wl], axis=1)
    p = jnp.dot(counts, m, preferred_element_type=jnp.float32)
    s = p[:, :EMB]
    square_of_sum = jnp.sum(s * s, axis=1, keepdims=True)
    sum_of_square = p[:, EMB:EMB + 1]
    lin_sparse = p[:, EMB + 1:EMB + 2]
    dense_lin = jnp.dot(dense_ref[...], wd_ref[...],
                        preferred_element_type=jnp.float32)
    out_ref[...] = (lin_sparse + dense_lin + b_ref[0, 0]
                    + 0.5 * (square_of_sum - sum_of_square))


def kernel(sparse_feat, dense_feat, linear_W, linear_b, lin_emb_tables,
           emb_tables):
    if sparse_feat.dtype != jnp.int32:
        sparse_feat = sparse_feat.astype(jnp.int32)
    counts = _sc_counts(sparse_feat)

    n_bins = N_FIELDS * FIELD
    emb_pad = jnp.zeros((N_BINS_PAD, EMB), jnp.float32)
    emb_pad = emb_pad.at[:n_bins].set(emb_tables.reshape(n_bins, EMB))
    lin_pad = jnp.zeros((N_BINS_PAD, 1), jnp.float32)
    lin_pad = lin_pad.at[:n_bins].set(lin_emb_tables.reshape(n_bins, 1))
    wf_pad = jnp.zeros((N_BINS_PAD, 1), jnp.float32)
    wf_pad = wf_pad.at[:n_bins].set(
        jnp.repeat(linear_W[0, :N_FIELDS], FIELD).reshape(n_bins, 1))
    wd = linear_W[0, N_FIELDS:].reshape(N_DENSE, 1)
    b = linear_b.reshape(1, 1)

    if dense_feat.dtype != jnp.float32:
        dense_feat = dense_feat.astype(jnp.float32)
    out = pl.pallas_call(
        _tc_body,
        out_shape=jax.ShapeDtypeStruct((BATCH, 1), jnp.float32),
    )(counts, emb_pad, lin_pad, wf_pad, dense_feat, wd, b)
    return out


# speedup vs baseline: 1.2090x; 1.2090x over previous
import functools

import jax
import jax.numpy as jnp
from jax import lax
from jax.experimental import pallas as pl
from jax.experimental.pallas import tpu as pltpu
from jax.experimental.pallas import tpu_sc as plsc

N_FIELDS = 26
FIELD = 50
STRIDE = 51
N_COLS = N_FIELDS * STRIDE
N_BINS_PAD = 1408
BATCH = 1024
N_DENSE = 13
EMB = 32

_NC = 2
_NS = 16
_NW = _NC * _NS
_ROWS_PER_W = BATCH // _NW
_GROUP = 16
_N_GROUPS = _ROWS_PER_W // _GROUP


_HIST_WORDS = _GROUP * N_BINS_PAD
_ZERO_UNROLL = 16


def _zero_hist(hist_v):
    zeros = jnp.zeros((16,), jnp.float32)

    @plsc.parallel_loop(0, N_BINS_PAD // 64, unroll=2)
    def body(i):
        base = i * 64
        for r in range(_GROUP):
            for k in range(4):
                hist_v[r, pl.ds(base + k * 16, 16)] = zeros


def _scatter_group(block_v, hist_v):
    lanes = lax.iota(jnp.int32, 16)
    row_base = lanes * N_COLS
    ones = jnp.ones((16,), jnp.float32)
    zero16 = jnp.zeros((16,), jnp.int32)

    def field_body(f, _):
        col0 = row_base + f * STRIDE
        bin0 = zero16 + f * FIELD

        @plsc.parallel_loop(0, FIELD, unroll=10)
        def col_body(j):
            vals = plsc.load_gather(block_v, [col0 + j])
            plsc.addupdate_scatter(hist_v, [lanes, bin0 + vals], ones)

        return 0
    lax.fori_loop(0, N_FIELDS, field_body, 0)


def _sc_counts_body(sparse_hbm, counts_hbm, block0, block1, hist0, hist1,
                    isem0, isem1, osem0, osem1):
    wid = lax.axis_index("s") * _NC + lax.axis_index("c")
    b0 = wid * _ROWS_PER_W
    b1 = b0 + _GROUP

    in0 = pltpu.async_copy(
        sparse_hbm.at[pl.ds(b0 * N_COLS, _GROUP * N_COLS)], block0, isem0)
    in1 = pltpu.async_copy(
        sparse_hbm.at[pl.ds(b1 * N_COLS, _GROUP * N_COLS)], block1, isem1)
    _zero_hist(hist0)
    _zero_hist(hist1)
    in0.wait()
    _scatter_group(block0, hist0)
    out0 = pltpu.async_copy(
        hist0, counts_hbm.at[pl.ds(b0, _GROUP), :], osem0)
    in1.wait()
    _scatter_group(block1, hist1)
    out1 = pltpu.async_copy(
        hist1, counts_hbm.at[pl.ds(b1, _GROUP), :], osem1)
    out0.wait()
    out1.wait()


def _sc_counts(sparse_2d):
    mesh = plsc.VectorSubcoreMesh(core_axis_name="c", subcore_axis_name="s")
    f = pl.kernel(
        _sc_counts_body,
        mesh=mesh,
        out_type=jax.ShapeDtypeStruct((BATCH, N_BINS_PAD), jnp.float32),
        scratch_types=[
            pltpu.VMEM((_GROUP * N_COLS,), jnp.int32),
            pltpu.VMEM((_GROUP * N_COLS,), jnp.int32),
            pltpu.VMEM((_GROUP, N_BINS_PAD), jnp.float32),
            pltpu.VMEM((_GROUP, N_BINS_PAD), jnp.float32),
            pltpu.SemaphoreType.DMA,
            pltpu.SemaphoreType.DMA,
            pltpu.SemaphoreType.DMA,
            pltpu.SemaphoreType.DMA,
        ],
        compiler_params=pltpu.CompilerParams(needs_layout_passes=False),
    )
    return f(sparse_2d)


def _tc_body(counts_ref, emb_ref, lin_ref, wf_ref, dense_ref, wd_ref, b_ref,
             out_ref):
    counts = counts_ref[...]
    emb = emb_ref[...]
    sq = jnp.sum(emb * emb, axis=1, keepdims=True)
    wl = lin_ref[...] * wf_ref[...]
    m = jnp.concatenate([emb, sq, wl], axis=1)
    p = jnp.dot(counts, m, preferred_element_type=jnp.float32)
    s = p[:, :EMB]
    square_of_sum = jnp.sum(s * s, axis=1, keepdims=True)
    sum_of_square = p[:, EMB:EMB + 1]
    lin_sparse = p[:, EMB + 1:EMB + 2]
    dense_lin = jnp.dot(dense_ref[...], wd_ref[...],
                        preferred_element_type=jnp.float32)
    out_ref[...] = (lin_sparse + dense_lin + b_ref[0, 0]
                    + 0.5 * (square_of_sum - sum_of_square))


def kernel(sparse_feat, dense_feat, linear_W, linear_b, lin_emb_tables,
           emb_tables):
    if sparse_feat.dtype != jnp.int32:
        sparse_feat = sparse_feat.astype(jnp.int32)
    counts = _sc_counts(sparse_feat.reshape(-1))

    n_bins = N_FIELDS * FIELD
    emb_pad = jnp.zeros((N_BINS_PAD, EMB), jnp.float32)
    emb_pad = emb_pad.at[:n_bins].set(emb_tables.reshape(n_bins, EMB))
    lin_pad = jnp.zeros((N_BINS_PAD, 1), jnp.float32)
    lin_pad = lin_pad.at[:n_bins].set(lin_emb_tables.reshape(n_bins, 1))
    wf_pad = jnp.zeros((N_BINS_PAD, 1), jnp.float32)
    wf_pad = wf_pad.at[:n_bins].set(
        jnp.repeat(linear_W[0, :N_FIELDS], FIELD).reshape(n_bins, 1))
    wd = linear_W[0, N_FIELDS:].reshape(N_DENSE, 1)
    b = linear_b.reshape(1, 1)

    if dense_feat.dtype != jnp.float32:
        dense_feat = dense_feat.astype(jnp.float32)
    out = pl.pallas_call(
        _tc_body,
        out_shape=jax.ShapeDtypeStruct((BATCH, 1), jnp.float32),
    )(counts, emb_pad, lin_pad, wf_pad, dense_feat, wd, b)
    return out
